# baseline (speedup 1.0000x reference)
import functools

import jax
import jax.numpy as jnp
from jax import lax
from jax.experimental import pallas as pl
from jax.experimental.pallas import tpu as pltpu
from jax.experimental.pallas import tpu_sc as plsc

_BLK = 2048


def _tables_body(x_ref, wn_ref, we1_ref, we2_ref, scal_ref,
                 s1_ref, s2_ref, alpha_ref):
    x = x_ref[...]
    b_n = scal_ref[0, 0]
    b_e = scal_ref[0, 1]
    r8 = s1_ref.shape[0]
    z = jnp.sum(x * wn_ref[...], axis=1) + b_n
    alpha_ref[...] = jax.nn.sigmoid(z).reshape(r8, 128)
    s1_ref[...] = (jnp.sum(x * we1_ref[...], axis=1) + b_e).reshape(r8, 128)
    s2_ref[...] = jnp.sum(x * we2_ref[...], axis=1).reshape(r8, 128)


def _run_tables(x, W_n, b_n, W_e, b_e):
    n, d = x.shape
    grid = (n + _BLK - 1) // _BLK
    npad = grid * _BLK
    r8 = _BLK // 128
    scalars = jnp.stack([b_n.astype(jnp.float32),
                         b_e.astype(jnp.float32)]).reshape(1, 2)
    tab_spec = pl.BlockSpec((r8, 128), lambda i: (i, 0))
    s1, s2, alpha2d = pl.pallas_call(
        _tables_body,
        grid=(grid,),
        in_specs=[pl.BlockSpec((_BLK, d), lambda i: (i, 0)),
                  pl.BlockSpec((1, d), lambda i: (0, 0)),
                  pl.BlockSpec((1, d), lambda i: (0, 0)),
                  pl.BlockSpec((1, d), lambda i: (0, 0)),
                  pl.BlockSpec(memory_space=pltpu.SMEM)],
        out_specs=[tab_spec, tab_spec, tab_spec],
        out_shape=[jax.ShapeDtypeStruct((npad // 128, 128), jnp.float32),
                   jax.ShapeDtypeStruct((npad // 128, 128), jnp.float32),
                   jax.ShapeDtypeStruct((npad // 128, 128), jnp.float32)],
    )(x, W_n.reshape(1, d), W_e[:d].reshape(1, d), W_e[d:].reshape(1, d),
      scalars)
    return (s1.reshape(npad), s2.reshape(npad), alpha2d.reshape(npad)[:n])


def _dense_body(grid, e, e_drop,
                x_ref, xnm_ref, wn_ref, pn_ref, scal_ref, eid_ref, ei_ref,
                xnode_ref, xedge_ref, eiout_ref, sem1, sem2):
    j = pl.program_id(0)
    cp1 = pltpu.make_async_copy(eid_ref, eiout_ref.at[:, pl.ds(0, e_drop)],
                                sem1)
    cp2 = pltpu.make_async_copy(ei_ref, eiout_ref.at[:, pl.ds(e_drop, e)],
                                sem2)

    @pl.when(j == 0)
    def _():
        cp1.start()
        cp2.start()

    x = x_ref[...]
    d = x.shape[1]
    b_n = scal_ref[0, 0]
    z = jnp.sum(x * wn_ref[...], axis=1, keepdims=True) + b_n
    alpha = jax.nn.sigmoid(z)
    xnode_ref[:, :d] = xnm_ref[...]
    xnode_ref[:, d:] = x + alpha * pn_ref[...]
    xedge_ref[:, :d] = x
    xedge_ref[:, d:] = x

    @pl.when(j == grid - 1)
    def _():
        cp1.wait()
        cp2.wait()


def _run_dense(x, x_node_masked, W_n, b_n, p_n,
               edge_index_dropped, edge_index_orig):
    n, d = x.shape
    e = edge_index_orig.shape[1]
    e_drop = edge_index_dropped.shape[1]
    grid = (n + _BLK - 1) // _BLK
    scalars = b_n.astype(jnp.float32).reshape(1, 1)
    row_spec = pl.BlockSpec((_BLK, d), lambda i: (i, 0))
    par_spec = pl.BlockSpec((1, d), lambda i: (0, 0))
    any_spec = pl.BlockSpec(memory_space=pl.ANY)
    return pl.pallas_call(
        functools.partial(_dense_body, grid, e, e_drop),
        grid=(grid,),
        in_specs=[row_spec, row_spec, par_spec, par_spec,
                  pl.BlockSpec(memory_space=pltpu.SMEM), any_spec, any_spec],
        out_specs=[pl.BlockSpec((_BLK, 2 * d), lambda i: (i, 0)),
                   pl.BlockSpec((_BLK, 2 * d), lambda i: (i, 0)),
                   any_spec],
        out_shape=[jax.ShapeDtypeStruct((n, 2 * d), jnp.float32),
                   jax.ShapeDtypeStruct((n, 2 * d), jnp.float32),
                   jax.ShapeDtypeStruct((2, e + e_drop), jnp.int32)],
        scratch_shapes=[pltpu.SemaphoreType.DMA, pltpu.SemaphoreType.DMA],
    )(x, x_node_masked, W_n.reshape(1, d), p_n.reshape(1, d), scalars,
      edge_index_dropped, edge_index_orig)


def _make_sc(npad, e, e_drop):
    info = plsc.get_sparse_core_info()
    nw = info.num_cores * info.num_subcores
    nc = info.num_cores
    pe_chunk = e // nw
    pd_chunk = e_drop // nw
    iters = pe_chunk // 16
    ones_n = ((pd_chunk + 15) // 16) * 16
    e_tot = e + e_drop
    mesh = plsc.VectorSubcoreMesh(core_axis_name="c", subcore_axis_name="s")

    @functools.partial(
        pl.kernel,
        mesh=mesh,
        compiler_params=pltpu.CompilerParams(needs_layout_passes=False),
        out_type=[jax.ShapeDtypeStruct((e,), jnp.float32),
                  jax.ShapeDtypeStruct((e_tot,), jnp.float32)],
        scratch_types=[pltpu.VMEM((npad,), jnp.float32),
                       pltpu.VMEM((npad,), jnp.float32),
                       pltpu.VMEM((pe_chunk,), jnp.int32),
                       pltpu.VMEM((pe_chunk,), jnp.int32),
                       pltpu.VMEM((pe_chunk,), jnp.float32),
                       pltpu.VMEM((pe_chunk,), jnp.float32),
                       pltpu.VMEM((16,), jnp.float32),
                       pltpu.VMEM((ones_n,), jnp.float32)],
    )
    def sc_kernel(s1_hbm, s2_hbm, ei_hbm, pe_hbm,
                  beta_hbm, ew_hbm,
                  s1_v, s2_v, src_v, dst_v, beta_v, w_v, pe_v, ones_v):
        wid = lax.axis_index("s") * nc + lax.axis_index("c")
        be = wid * pe_chunk
        bd = wid * pd_chunk

        pltpu.sync_copy(s1_hbm, s1_v)
        pltpu.sync_copy(s2_hbm, s2_v)
        pltpu.sync_copy(ei_hbm.at[pl.ds(be, pe_chunk)], src_v)
        pltpu.sync_copy(ei_hbm.at[pl.ds(e + be, pe_chunk)], dst_v)
        pltpu.sync_copy(pe_hbm, pe_v)
        p_e_vec = pe_v[...]

        @plsc.parallel_loop(0, iters, unroll=8)
        def _edge_loop(i):
            s = src_v[pl.ds(i * 16, 16)]
            t = dst_v[pl.ds(i * 16, 16)]
            a = plsc.load_gather(s1_v, [s])
            b = plsc.load_gather(s2_v, [t])
            beta = 1.0 / (1.0 + jnp.exp(-(a + b)))
            beta_v[pl.ds(i * 16, 16)] = beta
            w_v[pl.ds(i * 16, 16)] = 1.0 + beta * p_e_vec

        @plsc.parallel_loop(0, ones_n // 16, unroll=8)
        def _ones_loop(j):
            ones_v[pl.ds(j * 16, 16)] = jnp.ones((16,), jnp.float32)

        pltpu.sync_copy(beta_v, beta_hbm.at[pl.ds(be, pe_chunk)])
        pltpu.sync_copy(w_v, ew_hbm.at[pl.ds(e_drop + be, pe_chunk)])
        pltpu.sync_copy(ones_v.at[pl.ds(0, pd_chunk)],
                        ew_hbm.at[pl.ds(bd, pd_chunk)])

    return sc_kernel


def kernel(x, x_node_masked, edge_index_orig, edge_index_dropped,
           p_n, W_n, b_n, p_e, W_e, b_e):
    n, d = x.shape
    e = edge_index_orig.shape[1]
    e_drop = edge_index_dropped.shape[1]

    s1, s2, alpha = _run_tables(x, W_n, b_n, W_e, b_e)

    pe_vec = jnp.broadcast_to(jnp.reshape(p_e.astype(jnp.float32), (1,)), (16,))
    sc = _make_sc(s1.shape[0], e, e_drop)
    beta, edge_weight_edge = sc(s1, s2, edge_index_orig.reshape(-1), pe_vec)

    x_node, x_edge, edge_index_edge = _run_dense(
        x, x_node_masked, W_n, b_n, p_n, edge_index_dropped, edge_index_orig)

    return (x_node, x_edge, edge_weight_edge, alpha, beta, edge_index_edge)


# speedup vs baseline: 2.4043x; 2.4043x over previous
import functools

import jax
import jax.numpy as jnp
from jax import lax
from jax.experimental import pallas as pl
from jax.experimental.pallas import tpu as pltpu
from jax.experimental.pallas import tpu_sc as plsc

_BLK = 2048


def _tables_body(x_ref, wn_ref, we1_ref, we2_ref, scal_ref,
                 s1_ref, s2_ref, alpha_ref):
    x = x_ref[...]
    b_n = scal_ref[0, 0]
    b_e = scal_ref[0, 1]
    r8 = s1_ref.shape[0]
    z = jnp.sum(x * wn_ref[...], axis=1) + b_n
    alpha_ref[...] = jax.nn.sigmoid(z).reshape(r8, 128)
    s1_ref[...] = (jnp.sum(x * we1_ref[...], axis=1) + b_e).reshape(r8, 128)
    s2_ref[...] = jnp.sum(x * we2_ref[...], axis=1).reshape(r8, 128)


def _run_tables(x, W_n, b_n, W_e, b_e):
    n, d = x.shape
    grid = (n + _BLK - 1) // _BLK
    npad = grid * _BLK
    r8 = _BLK // 128
    scalars = jnp.stack([b_n.astype(jnp.float32),
                         b_e.astype(jnp.float32)]).reshape(1, 2)
    tab_spec = pl.BlockSpec((r8, 128), lambda i: (i, 0))
    s1, s2, alpha2d = pl.pallas_call(
        _tables_body,
        grid=(grid,),
        in_specs=[pl.BlockSpec((_BLK, d), lambda i: (i, 0)),
                  pl.BlockSpec((1, d), lambda i: (0, 0)),
                  pl.BlockSpec((1, d), lambda i: (0, 0)),
                  pl.BlockSpec((1, d), lambda i: (0, 0)),
                  pl.BlockSpec(memory_space=pltpu.SMEM)],
        out_specs=[tab_spec, tab_spec, tab_spec],
        out_shape=[jax.ShapeDtypeStruct((npad // 128, 128), jnp.float32),
                   jax.ShapeDtypeStruct((npad // 128, 128), jnp.float32),
                   jax.ShapeDtypeStruct((npad // 128, 128), jnp.float32)],
    )(x, W_n.reshape(1, d), W_e[:d].reshape(1, d), W_e[d:].reshape(1, d),
      scalars)
    return (s1.reshape(npad), s2.reshape(npad), alpha2d.reshape(npad)[:n])


def _dense_body(x_ref, xnm_ref, wn_ref, pn_ref, scal_ref,
                xnode_ref, xedge_ref):
    x = x_ref[...]
    d = x.shape[1]
    b_n = scal_ref[0, 0]
    z = jnp.sum(x * wn_ref[...], axis=1, keepdims=True) + b_n
    alpha = jax.nn.sigmoid(z)
    xnode_ref[:, :d] = xnm_ref[...]
    xnode_ref[:, d:] = x + alpha * pn_ref[...]
    xedge_ref[:, :d] = x
    xedge_ref[:, d:] = x


def _run_dense(x, x_node_masked, W_n, b_n, p_n):
    n, d = x.shape
    grid = (n + _BLK - 1) // _BLK
    scalars = b_n.astype(jnp.float32).reshape(1, 1)
    row_spec = pl.BlockSpec((_BLK, d), lambda i: (i, 0))
    par_spec = pl.BlockSpec((1, d), lambda i: (0, 0))
    return pl.pallas_call(
        _dense_body,
        grid=(grid,),
        in_specs=[row_spec, row_spec, par_spec, par_spec,
                  pl.BlockSpec(memory_space=pltpu.SMEM)],
        out_specs=[pl.BlockSpec((_BLK, 2 * d), lambda i: (i, 0)),
                   pl.BlockSpec((_BLK, 2 * d), lambda i: (i, 0))],
        out_shape=[jax.ShapeDtypeStruct((n, 2 * d), jnp.float32),
                   jax.ShapeDtypeStruct((n, 2 * d), jnp.float32)],
    )(x, x_node_masked, W_n.reshape(1, d), p_n.reshape(1, d), scalars)


def _concat_body(eid_ref, ei_ref, out_ref):
    j = pl.program_id(0)

    @pl.when(j < 10)
    def _():
        out_ref[...] = eid_ref[...]

    @pl.when(j >= 10)
    def _():
        out_ref[...] = ei_ref[...]


def _run_concat(edge_index_dropped, edge_index_orig):
    e = edge_index_orig.shape[1]
    e_drop = edge_index_dropped.shape[1]
    e_tot = e + e_drop
    cb = e_drop // 10
    grid = e_tot // cb
    return pl.pallas_call(
        _concat_body,
        grid=(grid,),
        in_specs=[pl.BlockSpec((2, cb), lambda j: (0, jnp.minimum(j, 9))),
                  pl.BlockSpec((2, cb), lambda j: (0, jnp.clip(j - 10, 0, 19)))],
        out_specs=pl.BlockSpec((2, cb), lambda j: (0, j)),
        out_shape=jax.ShapeDtypeStruct((2, e_tot), jnp.int32),
    )(edge_index_dropped, edge_index_orig)


def _make_sc(npad, e, e_drop):
    info = plsc.get_sparse_core_info()
    nw = info.num_cores * info.num_subcores
    nc = info.num_cores
    pe_chunk = e // nw
    pd_chunk = e_drop // nw
    iters = pe_chunk // 16
    ones_n = ((pd_chunk + 15) // 16) * 16
    e_tot = e + e_drop
    mesh = plsc.VectorSubcoreMesh(core_axis_name="c", subcore_axis_name="s")

    @functools.partial(
        pl.kernel,
        mesh=mesh,
        compiler_params=pltpu.CompilerParams(needs_layout_passes=False),
        out_type=[jax.ShapeDtypeStruct((e,), jnp.float32),
                  jax.ShapeDtypeStruct((e_tot,), jnp.float32)],
        scratch_types=[pltpu.VMEM((npad,), jnp.float32),
                       pltpu.VMEM((npad,), jnp.float32),
                       pltpu.VMEM((pe_chunk,), jnp.int32),
                       pltpu.VMEM((pe_chunk,), jnp.int32),
                       pltpu.VMEM((pe_chunk,), jnp.float32),
                       pltpu.VMEM((pe_chunk,), jnp.float32),
                       pltpu.VMEM((16,), jnp.float32),
                       pltpu.VMEM((ones_n,), jnp.float32)],
    )
    def sc_kernel(s1_hbm, s2_hbm, ei_hbm, pe_hbm,
                  beta_hbm, ew_hbm,
                  s1_v, s2_v, src_v, dst_v, beta_v, w_v, pe_v, ones_v):
        wid = lax.axis_index("s") * nc + lax.axis_index("c")
        be = wid * pe_chunk
        bd = wid * pd_chunk

        pltpu.sync_copy(s1_hbm, s1_v)
        pltpu.sync_copy(s2_hbm, s2_v)
        pltpu.sync_copy(ei_hbm.at[pl.ds(be, pe_chunk)], src_v)
        pltpu.sync_copy(ei_hbm.at[pl.ds(e + be, pe_chunk)], dst_v)
        pltpu.sync_copy(pe_hbm, pe_v)
        p_e_vec = pe_v[...]

        @plsc.parallel_loop(0, iters, unroll=8)
        def _edge_loop(i):
            s = src_v[pl.ds(i * 16, 16)]
            t = dst_v[pl.ds(i * 16, 16)]
            a = plsc.load_gather(s1_v, [s])
            b = plsc.load_gather(s2_v, [t])
            beta = 1.0 / (1.0 + jnp.exp(-(a + b)))
            beta_v[pl.ds(i * 16, 16)] = beta
            w_v[pl.ds(i * 16, 16)] = 1.0 + beta * p_e_vec

        @plsc.parallel_loop(0, ones_n // 16, unroll=8)
        def _ones_loop(j):
            ones_v[pl.ds(j * 16, 16)] = jnp.ones((16,), jnp.float32)

        pltpu.sync_copy(beta_v, beta_hbm.at[pl.ds(be, pe_chunk)])
        pltpu.sync_copy(w_v, ew_hbm.at[pl.ds(e_drop + be, pe_chunk)])
        pltpu.sync_copy(ones_v.at[pl.ds(0, pd_chunk)],
                        ew_hbm.at[pl.ds(bd, pd_chunk)])

    return sc_kernel


def kernel(x, x_node_masked, edge_index_orig, edge_index_dropped,
           p_n, W_n, b_n, p_e, W_e, b_e):
    n, d = x.shape
    e = edge_index_orig.shape[1]
    e_drop = edge_index_dropped.shape[1]

    s1, s2, alpha = _run_tables(x, W_n, b_n, W_e, b_e)

    pe_vec = jnp.broadcast_to(jnp.reshape(p_e.astype(jnp.float32), (1,)), (16,))
    sc = _make_sc(s1.shape[0], e, e_drop)
    beta, edge_weight_edge = sc(s1, s2, edge_index_orig.reshape(-1), pe_vec)

    edge_index_edge = _run_concat(edge_index_dropped, edge_index_orig)
    x_node, x_edge = _run_dense(x, x_node_masked, W_n, b_n, p_n)

    return (x_node, x_edge, edge_weight_edge, alpha, beta, edge_index_edge)


# speedup vs baseline: 2.8158x; 1.1711x over previous
import functools

import jax
import jax.numpy as jnp
from jax import lax
from jax.experimental import pallas as pl
from jax.experimental.pallas import tpu as pltpu
from jax.experimental.pallas import tpu_sc as plsc

_BLK = 2048


def _tables_body(x_ref, wn_ref, we1_ref, we2_ref, scal_ref,
                 s1_ref, s2_ref, alpha_ref):
    x = x_ref[...]
    b_n = scal_ref[0, 0]
    b_e = scal_ref[0, 1]
    r8 = s1_ref.shape[0]
    z = jnp.sum(x * wn_ref[...], axis=1) + b_n
    alpha_ref[...] = jax.nn.sigmoid(z).reshape(r8, 128)
    s1_ref[...] = (jnp.sum(x * we1_ref[...], axis=1) + b_e).reshape(r8, 128)
    s2_ref[...] = jnp.sum(x * we2_ref[...], axis=1).reshape(r8, 128)


def _run_tables(x, W_n, b_n, W_e, b_e):
    n, d = x.shape
    grid = (n + _BLK - 1) // _BLK
    npad = grid * _BLK
    r8 = _BLK // 128
    scalars = jnp.stack([b_n.astype(jnp.float32),
                         b_e.astype(jnp.float32)]).reshape(1, 2)
    tab_spec = pl.BlockSpec((r8, 128), lambda i: (i, 0))
    s1, s2, alpha2d = pl.pallas_call(
        _tables_body,
        grid=(grid,),
        in_specs=[pl.BlockSpec((_BLK, d), lambda i: (i, 0)),
                  pl.BlockSpec((1, d), lambda i: (0, 0)),
                  pl.BlockSpec((1, d), lambda i: (0, 0)),
                  pl.BlockSpec((1, d), lambda i: (0, 0)),
                  pl.BlockSpec(memory_space=pltpu.SMEM)],
        out_specs=[tab_spec, tab_spec, tab_spec],
        out_shape=[jax.ShapeDtypeStruct((npad // 128, 128), jnp.float32),
                   jax.ShapeDtypeStruct((npad // 128, 128), jnp.float32),
                   jax.ShapeDtypeStruct((npad // 128, 128), jnp.float32)],
    )(x, W_n.reshape(1, d), W_e[:d].reshape(1, d), W_e[d:].reshape(1, d),
      scalars)
    return (s1.reshape(npad), s2.reshape(npad), alpha2d.reshape(npad)[:n])


def _dense_body(x_ref, xnm_ref, wn_ref, pn_ref, scal_ref,
                xnode_ref, xedge_ref):
    x = x_ref[...]
    d = x.shape[1]
    b_n = scal_ref[0, 0]
    z = jnp.sum(x * wn_ref[...], axis=1, keepdims=True) + b_n
    alpha = jax.nn.sigmoid(z)
    xnode_ref[:, :d] = xnm_ref[...]
    xnode_ref[:, d:] = x + alpha * pn_ref[...]
    xedge_ref[:, :d] = x
    xedge_ref[:, d:] = x


def _run_dense(x, x_node_masked, W_n, b_n, p_n):
    n, d = x.shape
    grid = (n + _BLK - 1) // _BLK
    scalars = b_n.astype(jnp.float32).reshape(1, 1)
    row_spec = pl.BlockSpec((_BLK, d), lambda i: (i, 0))
    par_spec = pl.BlockSpec((1, d), lambda i: (0, 0))
    return pl.pallas_call(
        _dense_body,
        grid=(grid,),
        in_specs=[row_spec, row_spec, par_spec, par_spec,
                  pl.BlockSpec(memory_space=pltpu.SMEM)],
        out_specs=[pl.BlockSpec((_BLK, 2 * d), lambda i: (i, 0)),
                   pl.BlockSpec((_BLK, 2 * d), lambda i: (i, 0))],
        out_shape=[jax.ShapeDtypeStruct((n, 2 * d), jnp.float32),
                   jax.ShapeDtypeStruct((n, 2 * d), jnp.float32)],
    )(x, x_node_masked, W_n.reshape(1, d), p_n.reshape(1, d), scalars)


def _concat_body(nd, eid_ref, ei_ref, out_ref):
    j = pl.program_id(0)

    @pl.when(j < nd)
    def _():
        out_ref[...] = eid_ref[...]

    @pl.when(j >= nd)
    def _():
        out_ref[...] = ei_ref[...]


def _run_concat(edge_index_dropped, edge_index_orig):
    e = edge_index_orig.shape[1]
    e_drop = edge_index_dropped.shape[1]
    e_tot = e + e_drop
    cb = e_drop // 2
    grid = e_tot // cb
    nd = e_drop // cb
    return pl.pallas_call(
        functools.partial(_concat_body, nd),
        grid=(grid,),
        in_specs=[pl.BlockSpec((2, cb), lambda j: (0, jnp.minimum(j, nd - 1))),
                  pl.BlockSpec((2, cb),
                               lambda j: (0, jnp.clip(j - nd, 0, e // cb - 1)))],
        out_specs=pl.BlockSpec((2, cb), lambda j: (0, j)),
        out_shape=jax.ShapeDtypeStruct((2, e_tot), jnp.int32),
    )(edge_index_dropped, edge_index_orig)


def _make_sc(npad, e, e_drop):
    info = plsc.get_sparse_core_info()
    nw = info.num_cores * info.num_subcores
    nc = info.num_cores
    pe_chunk = e // nw
    pd_chunk = e_drop // nw
    iters = pe_chunk // 16
    ones_n = ((pd_chunk + 15) // 16) * 16
    e_tot = e + e_drop
    mesh = plsc.VectorSubcoreMesh(core_axis_name="c", subcore_axis_name="s")

    @functools.partial(
        pl.kernel,
        mesh=mesh,
        compiler_params=pltpu.CompilerParams(needs_layout_passes=False),
        out_type=[jax.ShapeDtypeStruct((e,), jnp.float32),
                  jax.ShapeDtypeStruct((e_tot,), jnp.float32)],
        scratch_types=[pltpu.VMEM((npad,), jnp.float32),
                       pltpu.VMEM((npad,), jnp.float32),
                       pltpu.VMEM((pe_chunk,), jnp.int32),
                       pltpu.VMEM((pe_chunk,), jnp.int32),
                       pltpu.VMEM((pe_chunk,), jnp.float32),
                       pltpu.VMEM((pe_chunk,), jnp.float32),
                       pltpu.VMEM((16,), jnp.float32),
                       pltpu.VMEM((ones_n,), jnp.float32)],
    )
    def sc_kernel(s1_hbm, s2_hbm, ei_hbm, pe_hbm,
                  beta_hbm, ew_hbm,
                  s1_v, s2_v, src_v, dst_v, beta_v, w_v, pe_v, ones_v):
        wid = lax.axis_index("s") * nc + lax.axis_index("c")
        be = wid * pe_chunk
        bd = wid * pd_chunk

        pltpu.sync_copy(s1_hbm, s1_v)
        pltpu.sync_copy(s2_hbm, s2_v)
        pltpu.sync_copy(ei_hbm.at[pl.ds(be, pe_chunk)], src_v)
        pltpu.sync_copy(ei_hbm.at[pl.ds(e + be, pe_chunk)], dst_v)
        pltpu.sync_copy(pe_hbm, pe_v)
        p_e_vec = pe_v[...]

        @plsc.parallel_loop(0, iters, unroll=8)
        def _edge_loop(i):
            s = src_v[pl.ds(i * 16, 16)]
            t = dst_v[pl.ds(i * 16, 16)]
            a = plsc.load_gather(s1_v, [s])
            b = plsc.load_gather(s2_v, [t])
            beta = 1.0 / (1.0 + jnp.exp(-(a + b)))
            beta_v[pl.ds(i * 16, 16)] = beta
            w_v[pl.ds(i * 16, 16)] = 1.0 + beta * p_e_vec

        @plsc.parallel_loop(0, ones_n // 16, unroll=8)
        def _ones_loop(j):
            ones_v[pl.ds(j * 16, 16)] = jnp.ones((16,), jnp.float32)

        pltpu.sync_copy(beta_v, beta_hbm.at[pl.ds(be, pe_chunk)])
        pltpu.sync_copy(w_v, ew_hbm.at[pl.ds(e_drop + be, pe_chunk)])
        pltpu.sync_copy(ones_v.at[pl.ds(0, pd_chunk)],
                        ew_hbm.at[pl.ds(bd, pd_chunk)])

    return sc_kernel


def kernel(x, x_node_masked, edge_index_orig, edge_index_dropped,
           p_n, W_n, b_n, p_e, W_e, b_e):
    n, d = x.shape
    e = edge_index_orig.shape[1]
    e_drop = edge_index_dropped.shape[1]

    s1, s2, alpha = _run_tables(x, W_n, b_n, W_e, b_e)

    pe_vec = jnp.broadcast_to(jnp.reshape(p_e.astype(jnp.float32), (1,)), (16,))
    sc = _make_sc(s1.shape[0], e, e_drop)
    beta, edge_weight_edge = sc(s1, s2, edge_index_orig.reshape(-1), pe_vec)

    edge_index_edge = _run_concat(edge_index_dropped, edge_index_orig)
    x_node, x_edge = _run_dense(x, x_node_masked, W_n, b_n, p_n)

    return (x_node, x_edge, edge_weight_edge, alpha, beta, edge_index_edge)


# speedup vs baseline: 3.1247x; 1.1097x over previous
import functools

import jax
import jax.numpy as jnp
from jax import lax
from jax.experimental import pallas as pl
from jax.experimental.pallas import tpu as pltpu
from jax.experimental.pallas import tpu_sc as plsc

_BLK = 2048


def _tables_body(x_ref, wn_ref, we1_ref, we2_ref, scal_ref,
                 s1_ref, s2_ref, alpha_ref):
    x = x_ref[...]
    b_n = scal_ref[0, 0]
    b_e = scal_ref[0, 1]
    r8 = s1_ref.shape[0]
    z = jnp.sum(x * wn_ref[...], axis=1) + b_n
    alpha_ref[...] = jax.nn.sigmoid(z).reshape(r8, 128)
    s1_ref[...] = (jnp.sum(x * we1_ref[...], axis=1) + b_e).reshape(r8, 128)
    s2_ref[...] = jnp.sum(x * we2_ref[...], axis=1).reshape(r8, 128)


def _run_tables(x, W_n, b_n, W_e, b_e):
    n, d = x.shape
    grid = (n + _BLK - 1) // _BLK
    npad = grid * _BLK
    r8 = _BLK // 128
    scalars = jnp.stack([b_n.astype(jnp.float32),
                         b_e.astype(jnp.float32)]).reshape(1, 2)
    tab_spec = pl.BlockSpec((r8, 128), lambda i: (i, 0))
    s1, s2, alpha2d = pl.pallas_call(
        _tables_body,
        grid=(grid,),
        in_specs=[pl.BlockSpec((_BLK, d), lambda i: (i, 0)),
                  pl.BlockSpec((1, d), lambda i: (0, 0)),
                  pl.BlockSpec((1, d), lambda i: (0, 0)),
                  pl.BlockSpec((1, d), lambda i: (0, 0)),
                  pl.BlockSpec(memory_space=pltpu.SMEM)],
        out_specs=[tab_spec, tab_spec, tab_spec],
        out_shape=[jax.ShapeDtypeStruct((npad // 128, 128), jnp.float32),
                   jax.ShapeDtypeStruct((npad // 128, 128), jnp.float32),
                   jax.ShapeDtypeStruct((npad // 128, 128), jnp.float32)],
    )(x, W_n.reshape(1, d), W_e[:d].reshape(1, d), W_e[d:].reshape(1, d),
      scalars)
    return (s1.reshape(npad), s2.reshape(npad), alpha2d.reshape(npad)[:n])


def _dense_body(x_ref, xnm_ref, wn_ref, pn_ref, scal_ref,
                xnode_ref, xedge_ref):
    x = x_ref[...]
    d = x.shape[1]
    b_n = scal_ref[0, 0]
    z = jnp.sum(x * wn_ref[...], axis=1, keepdims=True) + b_n
    alpha = jax.nn.sigmoid(z)
    xnode_ref[:, :d] = xnm_ref[...]
    xnode_ref[:, d:] = x + alpha * pn_ref[...]
    xedge_ref[:, :d] = x
    xedge_ref[:, d:] = x


def _run_dense(x, x_node_masked, W_n, b_n, p_n):
    n, d = x.shape
    grid = (n + _BLK - 1) // _BLK
    scalars = b_n.astype(jnp.float32).reshape(1, 1)
    row_spec = pl.BlockSpec((_BLK, d), lambda i: (i, 0))
    par_spec = pl.BlockSpec((1, d), lambda i: (0, 0))
    return pl.pallas_call(
        _dense_body,
        grid=(grid,),
        in_specs=[row_spec, row_spec, par_spec, par_spec,
                  pl.BlockSpec(memory_space=pltpu.SMEM)],
        out_specs=[pl.BlockSpec((_BLK, 2 * d), lambda i: (i, 0)),
                   pl.BlockSpec((_BLK, 2 * d), lambda i: (i, 0))],
        out_shape=[jax.ShapeDtypeStruct((n, 2 * d), jnp.float32),
                   jax.ShapeDtypeStruct((n, 2 * d), jnp.float32)],
    )(x, x_node_masked, W_n.reshape(1, d), p_n.reshape(1, d), scalars)


def _concat_body(nd, eid_ref, ei_ref, out_ref):
    j = pl.program_id(0)

    @pl.when(j < nd)
    def _():
        out_ref[...] = eid_ref[...]

    @pl.when(j >= nd)
    def _():
        out_ref[...] = ei_ref[...]


def _run_concat(edge_index_dropped, edge_index_orig):
    e = edge_index_orig.shape[1]
    e_drop = edge_index_dropped.shape[1]
    e_tot = e + e_drop
    cb = e_drop // 2
    grid = e_tot // cb
    nd = e_drop // cb
    return pl.pallas_call(
        functools.partial(_concat_body, nd),
        grid=(grid,),
        in_specs=[pl.BlockSpec((2, cb), lambda j: (0, jnp.minimum(j, nd - 1))),
                  pl.BlockSpec((2, cb),
                               lambda j: (0, jnp.clip(j - nd, 0, e // cb - 1)))],
        out_specs=pl.BlockSpec((2, cb), lambda j: (0, j)),
        out_shape=jax.ShapeDtypeStruct((2, e_tot), jnp.int32),
    )(edge_index_dropped, edge_index_orig)


def _make_sc(npad, e, e_drop):
    info = plsc.get_sparse_core_info()
    nw = info.num_cores * info.num_subcores
    nc = info.num_cores
    tiles = e // 128
    tpw = tiles // nw
    xtra = tiles - tpw * nw
    pe_chunk = tpw * 128
    pd_chunk = e_drop // nw
    iters = pe_chunk // 16
    ones_n = ((pd_chunk + 15) // 16) * 16
    e_tot = e + e_drop
    mesh = plsc.VectorSubcoreMesh(core_axis_name="c", subcore_axis_name="s")

    @functools.partial(
        pl.kernel,
        mesh=mesh,
        compiler_params=pltpu.CompilerParams(needs_layout_passes=False),
        out_type=[jax.ShapeDtypeStruct((e,), jnp.float32),
                  jax.ShapeDtypeStruct((e_tot,), jnp.float32)],
        scratch_types=[pltpu.VMEM((npad,), jnp.float32),
                       pltpu.VMEM((npad,), jnp.float32),
                       pltpu.VMEM((2, pe_chunk), jnp.int32),
                       pltpu.VMEM((2, 128), jnp.int32),
                       pltpu.VMEM((pe_chunk,), jnp.float32),
                       pltpu.VMEM((pe_chunk,), jnp.float32),
                       pltpu.VMEM((128,), jnp.float32),
                       pltpu.VMEM((128,), jnp.float32),
                       pltpu.VMEM((16,), jnp.float32),
                       pltpu.VMEM((ones_n,), jnp.float32)],
    )
    def sc_kernel(s1_hbm, s2_hbm, ei_hbm, pe_hbm,
                  beta_hbm, ew_hbm,
                  s1_v, s2_v, ei_v, eix_v, beta_v, w_v, betax_v, wx_v,
                  pe_v, ones_v):
        wid = lax.axis_index("s") * nc + lax.axis_index("c")
        be = wid * pe_chunk
        bd = wid * pd_chunk

        pltpu.sync_copy(s1_hbm, s1_v)
        pltpu.sync_copy(s2_hbm, s2_v)
        pltpu.sync_copy(ei_hbm.at[:, pl.ds(be, pe_chunk)], ei_v)
        pltpu.sync_copy(pe_hbm, pe_v)
        p_e_vec = pe_v[...]

        @plsc.parallel_loop(0, iters, unroll=8)
        def _edge_loop(i):
            s = ei_v[0, pl.ds(i * 16, 16)]
            t = ei_v[1, pl.ds(i * 16, 16)]
            a = plsc.load_gather(s1_v, [s])
            b = plsc.load_gather(s2_v, [t])
            beta = 1.0 / (1.0 + jnp.exp(-(a + b)))
            beta_v[pl.ds(i * 16, 16)] = beta
            w_v[pl.ds(i * 16, 16)] = 1.0 + beta * p_e_vec

        @plsc.parallel_loop(0, ones_n // 16, unroll=8)
        def _ones_loop(j):
            ones_v[pl.ds(j * 16, 16)] = jnp.ones((16,), jnp.float32)

        pltpu.sync_copy(beta_v, beta_hbm.at[pl.ds(be, pe_chunk)])
        pltpu.sync_copy(w_v, ew_hbm.at[pl.ds(e_drop + be, pe_chunk)])
        pltpu.sync_copy(ones_v.at[pl.ds(0, pd_chunk)],
                        ew_hbm.at[pl.ds(bd, pd_chunk)])

        @pl.when(wid < xtra)
        def _():
            bx = nw * pe_chunk + wid * 128
            pltpu.sync_copy(ei_hbm.at[:, pl.ds(bx, 128)], eix_v)

            @plsc.parallel_loop(0, 128 // 16, unroll=8)
            def _tail_loop(i):
                s = eix_v[0, pl.ds(i * 16, 16)]
                t = eix_v[1, pl.ds(i * 16, 16)]
                a = plsc.load_gather(s1_v, [s])
                b = plsc.load_gather(s2_v, [t])
                beta = 1.0 / (1.0 + jnp.exp(-(a + b)))
                betax_v[pl.ds(i * 16, 16)] = beta
                wx_v[pl.ds(i * 16, 16)] = 1.0 + beta * p_e_vec

            pltpu.sync_copy(betax_v, beta_hbm.at[pl.ds(bx, 128)])
            pltpu.sync_copy(wx_v, ew_hbm.at[pl.ds(e_drop + bx, 128)])

    return sc_kernel


def kernel(x, x_node_masked, edge_index_orig, edge_index_dropped,
           p_n, W_n, b_n, p_e, W_e, b_e):
    n, d = x.shape
    e = edge_index_orig.shape[1]
    e_drop = edge_index_dropped.shape[1]

    s1, s2, alpha = _run_tables(x, W_n, b_n, W_e, b_e)

    pe_vec = jnp.broadcast_to(jnp.reshape(p_e.astype(jnp.float32), (1,)), (16,))
    sc = _make_sc(s1.shape[0], e, e_drop)
    beta, edge_weight_edge = sc(s1, s2, edge_index_orig, pe_vec)

    edge_index_edge = _run_concat(edge_index_dropped, edge_index_orig)
    x_node, x_edge = _run_dense(x, x_node_masked, W_n, b_n, p_n)

    return (x_node, x_edge, edge_weight_edge, alpha, beta, edge_index_edge)


# speedup vs baseline: 3.1439x; 1.0061x over previous
import functools

import jax
import jax.numpy as jnp
from jax import lax
from jax.experimental import pallas as pl
from jax.experimental.pallas import tpu as pltpu
from jax.experimental.pallas import tpu_sc as plsc

_BLK = 2048


def _tables_body(x_ref, wn_ref, we1_ref, we2_ref, scal_ref,
                 s1_ref, s2_ref, alpha_ref):
    x = x_ref[...]
    b_n = scal_ref[0, 0]
    b_e = scal_ref[0, 1]
    r8 = s1_ref.shape[0]
    z = jnp.sum(x * wn_ref[...], axis=1) + b_n
    alpha_ref[...] = jax.nn.sigmoid(z).reshape(r8, 128)
    s1_ref[...] = (jnp.sum(x * we1_ref[...], axis=1) + b_e).reshape(r8, 128)
    s2_ref[...] = jnp.sum(x * we2_ref[...], axis=1).reshape(r8, 128)


def _run_tables(x, W_n, b_n, W_e, b_e):
    n, d = x.shape
    grid = (n + _BLK - 1) // _BLK
    npad = grid * _BLK
    r8 = _BLK // 128
    scalars = jnp.stack([b_n.astype(jnp.float32),
                         b_e.astype(jnp.float32)]).reshape(1, 2)
    tab_spec = pl.BlockSpec((r8, 128), lambda i: (i, 0))
    s1, s2, alpha2d = pl.pallas_call(
        _tables_body,
        grid=(grid,),
        in_specs=[pl.BlockSpec((_BLK, d), lambda i: (i, 0)),
                  pl.BlockSpec((1, d), lambda i: (0, 0)),
                  pl.BlockSpec((1, d), lambda i: (0, 0)),
                  pl.BlockSpec((1, d), lambda i: (0, 0)),
                  pl.BlockSpec(memory_space=pltpu.SMEM)],
        out_specs=[tab_spec, tab_spec, tab_spec],
        out_shape=[jax.ShapeDtypeStruct((npad // 128, 128), jnp.float32),
                   jax.ShapeDtypeStruct((npad // 128, 128), jnp.float32),
                   jax.ShapeDtypeStruct((npad // 128, 128), jnp.float32)],
    )(x, W_n.reshape(1, d), W_e[:d].reshape(1, d), W_e[d:].reshape(1, d),
      scalars)
    return (s1.reshape(npad), s2.reshape(npad), alpha2d.reshape(npad)[:n])


def _dense_body(x_ref, xnm_ref, wn_ref, pn_ref, scal_ref,
                xnode_ref, xedge_ref):
    x = x_ref[...]
    d = x.shape[1]
    b_n = scal_ref[0, 0]
    z = jnp.sum(x * wn_ref[...], axis=1, keepdims=True) + b_n
    alpha = jax.nn.sigmoid(z)
    xnode_ref[:, :d] = xnm_ref[...]
    xnode_ref[:, d:] = x + alpha * pn_ref[...]
    xedge_ref[:, :d] = x
    xedge_ref[:, d:] = x


def _run_dense(x, x_node_masked, W_n, b_n, p_n):
    n, d = x.shape
    grid = (n + _BLK - 1) // _BLK
    scalars = b_n.astype(jnp.float32).reshape(1, 1)
    row_spec = pl.BlockSpec((_BLK, d), lambda i: (i, 0))
    par_spec = pl.BlockSpec((1, d), lambda i: (0, 0))
    return pl.pallas_call(
        _dense_body,
        grid=(grid,),
        in_specs=[row_spec, row_spec, par_spec, par_spec,
                  pl.BlockSpec(memory_space=pltpu.SMEM)],
        out_specs=[pl.BlockSpec((_BLK, 2 * d), lambda i: (i, 0)),
                   pl.BlockSpec((_BLK, 2 * d), lambda i: (i, 0))],
        out_shape=[jax.ShapeDtypeStruct((n, 2 * d), jnp.float32),
                   jax.ShapeDtypeStruct((n, 2 * d), jnp.float32)],
    )(x, x_node_masked, W_n.reshape(1, d), p_n.reshape(1, d), scalars)


def _concat_body(nd, eid_ref, ei_ref, out_ref):
    j = pl.program_id(0)

    @pl.when(j < nd)
    def _():
        out_ref[...] = eid_ref[...]

    @pl.when(j >= nd)
    def _():
        out_ref[...] = ei_ref[...]


def _run_concat(edge_index_dropped, edge_index_orig):
    e = edge_index_orig.shape[1]
    e_drop = edge_index_dropped.shape[1]
    e_tot = e + e_drop
    cb = e_drop // 2
    grid = e_tot // cb
    nd = e_drop // cb
    return pl.pallas_call(
        functools.partial(_concat_body, nd),
        grid=(grid,),
        in_specs=[pl.BlockSpec((2, cb), lambda j: (0, jnp.minimum(j, nd - 1))),
                  pl.BlockSpec((2, cb),
                               lambda j: (0, jnp.clip(j - nd, 0, e // cb - 1)))],
        out_specs=pl.BlockSpec((2, cb), lambda j: (0, j)),
        out_shape=jax.ShapeDtypeStruct((2, e_tot), jnp.int32),
    )(edge_index_dropped, edge_index_orig)


def _make_sc(npad, e, e_drop):
    info = plsc.get_sparse_core_info()
    nw = info.num_cores * info.num_subcores
    nc = info.num_cores
    tiles = e // 128
    tpw = tiles // nw
    xtra = tiles - tpw * nw
    pe_chunk = tpw * 128
    pc_x = pe_chunk + 128
    pd_chunk = e_drop // nw
    iters = pc_x // 16
    ones_n = ((pd_chunk + 15) // 16) * 16
    e_tot = e + e_drop
    mesh = plsc.VectorSubcoreMesh(core_axis_name="c", subcore_axis_name="s")

    @functools.partial(
        pl.kernel,
        mesh=mesh,
        compiler_params=pltpu.CompilerParams(needs_layout_passes=False),
        out_type=[jax.ShapeDtypeStruct((e,), jnp.float32),
                  jax.ShapeDtypeStruct((e_tot,), jnp.float32)],
        scratch_types=[pltpu.VMEM((npad,), jnp.float32),
                       pltpu.VMEM((npad,), jnp.float32),
                       pltpu.VMEM((2, pc_x), jnp.int32),
                       pltpu.VMEM((pc_x,), jnp.float32),
                       pltpu.VMEM((pc_x,), jnp.float32),
                       pltpu.VMEM((16,), jnp.float32),
                       pltpu.VMEM((ones_n,), jnp.float32)],
    )
    def sc_kernel(s1_hbm, s2_hbm, ei_hbm, pe_hbm,
                  beta_hbm, ew_hbm,
                  s1_v, s2_v, ei_v, beta_v, w_v, pe_v, ones_v):
        wid = lax.axis_index("s") * nc + lax.axis_index("c")
        be = wid * pe_chunk
        bd = wid * pd_chunk

        pltpu.sync_copy(s1_hbm, s1_v)
        pltpu.sync_copy(s2_hbm, s2_v)
        pltpu.sync_copy(ei_hbm.at[:, pl.ds(be, pe_chunk)],
                        ei_v.at[:, pl.ds(0, pe_chunk)])
        zero = jnp.zeros((16,), jnp.int32)
        for q in range(128 // 16):
            ei_v[0, pl.ds(pe_chunk + q * 16, 16)] = zero
            ei_v[1, pl.ds(pe_chunk + q * 16, 16)] = zero
        bx = nw * pe_chunk + wid * 128

        @pl.when(wid < xtra)
        def _():
            pltpu.sync_copy(ei_hbm.at[:, pl.ds(bx, 128)],
                            ei_v.at[:, pl.ds(pe_chunk, 128)])

        pltpu.sync_copy(pe_hbm, pe_v)
        p_e_vec = pe_v[...]

        @plsc.parallel_loop(0, iters, unroll=4)
        def _edge_loop(i):
            s = ei_v[0, pl.ds(i * 16, 16)]
            t = ei_v[1, pl.ds(i * 16, 16)]
            a = plsc.load_gather(s1_v, [s])
            b = plsc.load_gather(s2_v, [t])
            beta = 1.0 / (1.0 + jnp.exp(-(a + b)))
            beta_v[pl.ds(i * 16, 16)] = beta
            w_v[pl.ds(i * 16, 16)] = 1.0 + beta * p_e_vec

        @plsc.parallel_loop(0, ones_n // 16, unroll=4)
        def _ones_loop(j):
            ones_v[pl.ds(j * 16, 16)] = jnp.ones((16,), jnp.float32)

        pltpu.sync_copy(beta_v.at[pl.ds(0, pe_chunk)],
                        beta_hbm.at[pl.ds(be, pe_chunk)])
        pltpu.sync_copy(w_v.at[pl.ds(0, pe_chunk)],
                        ew_hbm.at[pl.ds(e_drop + be, pe_chunk)])
        pltpu.sync_copy(ones_v.at[pl.ds(0, pd_chunk)],
                        ew_hbm.at[pl.ds(bd, pd_chunk)])

        @pl.when(wid < xtra)
        def _():
            pltpu.sync_copy(beta_v.at[pl.ds(pe_chunk, 128)],
                            beta_hbm.at[pl.ds(bx, 128)])
            pltpu.sync_copy(w_v.at[pl.ds(pe_chunk, 128)],
                            ew_hbm.at[pl.ds(e_drop + bx, 128)])

    return sc_kernel


def kernel(x, x_node_masked, edge_index_orig, edge_index_dropped,
           p_n, W_n, b_n, p_e, W_e, b_e):
    n, d = x.shape
    e = edge_index_orig.shape[1]
    e_drop = edge_index_dropped.shape[1]

    s1, s2, alpha = _run_tables(x, W_n, b_n, W_e, b_e)

    pe_vec = jnp.broadcast_to(jnp.reshape(p_e.astype(jnp.float32), (1,)), (16,))
    sc = _make_sc(s1.shape[0], e, e_drop)
    beta, edge_weight_edge = sc(s1, s2, edge_index_orig, pe_vec)

    edge_index_edge = _run_concat(edge_index_dropped, edge_index_orig)
    x_node, x_edge = _run_dense(x, x_node_masked, W_n, b_n, p_n)

    return (x_node, x_edge, edge_weight_edge, alpha, beta, edge_index_edge)
